# baseline (speedup 1.0000x reference)
import functools

import jax
import jax.numpy as jnp
from jax import lax
from jax.experimental import pallas as pl
from jax.experimental.pallas import tpu as pltpu
from jax.experimental.pallas import tpu_sc as plsc

_N = 32768 * 1024
_NC = 2
_NS = 16
_NW = _NC * _NS
_PER_W = _N // _NW
_CHUNK = 32768
_NCHUNK = _PER_W // _CHUNK
_LANES = 16

def _sc_body(x_hbm, y_hbm, out_hbm, buf0, buf1, accv, sem0, sem1):
    c = lax.axis_index("c")
    s = lax.axis_index("s")
    wid = s * _NC + c
    base = wid * _PER_W
    bufs = (buf0, buf1)
    sems = (sem0, sem1)
    ones = jnp.ones((_LANES,), jnp.int32)
    zeros = jnp.zeros((_LANES,), jnp.int32)

    for oidx, arr in enumerate((x_hbm, y_hbm)):
        pltpu.async_copy(arr.at[pl.ds(base, _CHUNK)], buf0, sem0)
        pltpu.async_copy(arr.at[pl.ds(base + _CHUNK, _CHUNK)], buf1, sem1)

        def pair_body(g, acc, arr=arr):
            for b in range(2):
                buf, sem = bufs[b], sems[b]
                pltpu.make_async_copy(arr.at[pl.ds(base, _CHUNK)], buf, sem).wait()

                @plsc.parallel_loop(0, _CHUNK, step=_LANES, unroll=8, carry=acc)
                def inner(i, a, buf=buf):
                    v = buf[pl.ds(i, _LANES)]
                    return a + jnp.where(v > 0.0, ones, zeros)

                acc = inner
                nk = g * 2 + b + 2

                @pl.when(nk < _NCHUNK)
                def _(arr=arr, buf=buf, sem=sem, nk=nk):
                    pltpu.async_copy(
                        arr.at[pl.ds(base + nk * _CHUNK, _CHUNK)], buf, sem
                    )

            return acc

        acc = lax.fori_loop(
            0, _NCHUNK // 2, pair_body, jnp.zeros((_LANES,), jnp.int32)
        )
        accv[...] = acc
        pltpu.sync_copy(accv, out_hbm.at[wid, oidx])


_sc_count = functools.partial(
    pl.kernel,
    out_type=jax.ShapeDtypeStruct((_NW, 2, _LANES), jnp.int32),
    mesh=plsc.VectorSubcoreMesh(core_axis_name="c", subcore_axis_name="s"),
    scratch_types=[
        pltpu.VMEM((_CHUNK,), jnp.float32),
        pltpu.VMEM((_CHUNK,), jnp.float32),
        pltpu.VMEM((_LANES,), jnp.int32),
        pltpu.SemaphoreType.DMA,
        pltpu.SemaphoreType.DMA,
    ],
)(_sc_body)


def kernel(x, y):
    parts = _sc_count(x.reshape(-1), y.reshape(-1))
    totals = parts.sum(axis=(0, 2))
    return jnp.maximum(totals[0], totals[1])


# speedup vs baseline: 6.0672x; 6.0672x over previous
import functools

import jax
import jax.numpy as jnp
from jax import lax
from jax.experimental import pallas as pl
from jax.experimental.pallas import tpu as pltpu
from jax.experimental.pallas import tpu_sc as plsc

_ROWS = 32768
_COLS = 1024
_NC = 2
_NS = 16
_NW = _NC * _NS
_ROWS_W = _ROWS // _NW
_CROWS = 32
_NCHUNK = _ROWS_W // _CROWS
_LANES = 16
_CSTEPS = _COLS // _LANES
_NACC = 8


def _sc_body(x_hbm, y_hbm, out_hbm, buf0, buf1, accv, sem0, sem1):
    c = lax.axis_index("c")
    s = lax.axis_index("s")
    wid = s * _NC + c
    rbase = wid * _ROWS_W
    bufs = (buf0, buf1)
    sems = (sem0, sem1)
    ones = jnp.ones((_LANES,), jnp.int32)
    zeros = jnp.zeros((_LANES,), jnp.int32)
    zero_accs = (jnp.zeros((_LANES,), jnp.int32),) * _NACC

    for oidx, arr in enumerate((x_hbm, y_hbm)):
        pltpu.async_copy(arr.at[pl.ds(rbase, _CROWS), :], buf0, sem0)
        pltpu.async_copy(arr.at[pl.ds(rbase + _CROWS, _CROWS), :], buf1, sem1)

        def pair_body(g, accs, arr=arr):
            for b in range(2):
                buf, sem = bufs[b], sems[b]
                pltpu.make_async_copy(
                    arr.at[pl.ds(rbase, _CROWS), :], buf, sem
                ).wait()

                @plsc.parallel_loop(0, _CROWS, step=1, carry=accs)
                def row_loop(r, a, buf=buf):
                    a = list(a)
                    for j in range(_CSTEPS):
                        v = buf[r, pl.ds(j * _LANES, _LANES)]
                        a[j % _NACC] = a[j % _NACC] + jnp.where(
                            v > 0.0, ones, zeros
                        )
                    return tuple(a)

                accs = row_loop
                nk = g * 2 + b + 2

                @pl.when(nk < _NCHUNK)
                def _(arr=arr, buf=buf, sem=sem, nk=nk):
                    pltpu.async_copy(
                        arr.at[pl.ds(rbase + nk * _CROWS, _CROWS), :], buf, sem
                    )

            return accs

        accs = lax.fori_loop(0, _NCHUNK // 2, pair_body, zero_accs)
        total = accs[0]
        for a in accs[1:]:
            total = total + a
        accv[...] = total
        pltpu.sync_copy(accv, out_hbm.at[wid, oidx])


_sc_count = functools.partial(
    pl.kernel,
    out_type=jax.ShapeDtypeStruct((_NW, 2, _LANES), jnp.int32),
    mesh=plsc.VectorSubcoreMesh(core_axis_name="c", subcore_axis_name="s"),
    scratch_types=[
        pltpu.VMEM((_CROWS, _COLS), jnp.float32),
        pltpu.VMEM((_CROWS, _COLS), jnp.float32),
        pltpu.VMEM((_LANES,), jnp.int32),
        pltpu.SemaphoreType.DMA,
        pltpu.SemaphoreType.DMA,
    ],
)(_sc_body)


def kernel(x, y):
    parts = _sc_count(x, y)
    totals = parts.sum(axis=(0, 2))
    return jnp.maximum(totals[0], totals[1])


# speedup vs baseline: 7.8574x; 1.2950x over previous
import functools

import jax
import jax.numpy as jnp
from jax import lax
from jax.experimental import pallas as pl
from jax.experimental.pallas import tpu as pltpu
from jax.experimental.pallas import tpu_sc as plsc

_ROWS = 32768
_COLS = 1024

_SC_ROWS = 12288
_NC = 2
_NS = 16
_NW = _NC * _NS
_ROWS_W = _SC_ROWS // _NW
_CROWS = 32
_NCHUNK = _ROWS_W // _CROWS
_LANES = 16
_CSTEPS = _COLS // _LANES
_NACC = 8

_TC_BLOCK = 1024
_TC_OFF = _SC_ROWS // _TC_BLOCK
_TC_GRID = (_ROWS - _SC_ROWS) // _TC_BLOCK


def _sc_body(x_hbm, y_hbm, out_hbm, buf0, buf1, accv, sem0, sem1):
    c = lax.axis_index("c")
    s = lax.axis_index("s")
    wid = s * _NC + c
    rbase = wid * _ROWS_W
    bufs = (buf0, buf1)
    sems = (sem0, sem1)
    ones = jnp.ones((_LANES,), jnp.int32)
    zeros = jnp.zeros((_LANES,), jnp.int32)
    zero_accs = (jnp.zeros((_LANES,), jnp.int32),) * _NACC

    for oidx, arr in enumerate((x_hbm, y_hbm)):
        pltpu.async_copy(arr.at[pl.ds(rbase, _CROWS), :], buf0, sem0)
        pltpu.async_copy(arr.at[pl.ds(rbase + _CROWS, _CROWS), :], buf1, sem1)

        def pair_body(g, accs, arr=arr):
            for b in range(2):
                buf, sem = bufs[b], sems[b]
                pltpu.make_async_copy(
                    arr.at[pl.ds(rbase, _CROWS), :], buf, sem
                ).wait()

                @plsc.parallel_loop(0, _CROWS, step=1, carry=accs)
                def row_loop(r, a, buf=buf):
                    a = list(a)
                    for j in range(_CSTEPS):
                        v = buf[r, pl.ds(j * _LANES, _LANES)]
                        a[j % _NACC] = a[j % _NACC] + jnp.where(
                            v > 0.0, ones, zeros
                        )
                    return tuple(a)

                accs = row_loop
                nk = g * 2 + b + 2

                @pl.when(nk < _NCHUNK)
                def _(arr=arr, buf=buf, sem=sem, nk=nk):
                    pltpu.async_copy(
                        arr.at[pl.ds(rbase + nk * _CROWS, _CROWS), :], buf, sem
                    )

            return accs

        accs = lax.fori_loop(0, _NCHUNK // 2, pair_body, zero_accs)
        total = accs[0]
        for a in accs[1:]:
            total = total + a
        accv[...] = total
        pltpu.sync_copy(accv, out_hbm.at[wid, oidx])


_sc_count = functools.partial(
    pl.kernel,
    out_type=jax.ShapeDtypeStruct((_NW, 2, _LANES), jnp.int32),
    mesh=plsc.VectorSubcoreMesh(core_axis_name="c", subcore_axis_name="s"),
    scratch_types=[
        pltpu.VMEM((_CROWS, _COLS), jnp.float32),
        pltpu.VMEM((_CROWS, _COLS), jnp.float32),
        pltpu.VMEM((_LANES,), jnp.int32),
        pltpu.SemaphoreType.DMA,
        pltpu.SemaphoreType.DMA,
    ],
)(_sc_body)


def _tc_body(x_ref, y_ref, out_ref, acc_ref):
    i = pl.program_id(0)

    @pl.when(i == 0)
    def _init():
        acc_ref[0] = 0
        acc_ref[1] = 0

    acc_ref[0] += jnp.sum((x_ref[...] > 0).astype(jnp.int32))
    acc_ref[1] += jnp.sum((y_ref[...] > 0).astype(jnp.int32))

    @pl.when(i == _TC_GRID - 1)
    def _finish():
        out_ref[0] = acc_ref[0]
        out_ref[1] = acc_ref[1]


def _tc_count(x, y):
    return pl.pallas_call(
        _tc_body,
        grid=(_TC_GRID,),
        in_specs=[
            pl.BlockSpec((_TC_BLOCK, _COLS), lambda i: (i + _TC_OFF, 0)),
            pl.BlockSpec((_TC_BLOCK, _COLS), lambda i: (i + _TC_OFF, 0)),
        ],
        out_specs=pl.BlockSpec(memory_space=pltpu.SMEM),
        out_shape=jax.ShapeDtypeStruct((2,), jnp.int32),
        scratch_shapes=[pltpu.SMEM((2,), jnp.int32)],
    )(x, y)


def kernel(x, y):
    sc_parts = _sc_count(x, y)
    tc_counts = _tc_count(x, y)
    sc_totals = sc_parts.sum(axis=(0, 2))
    totals = sc_totals + tc_counts
    return jnp.maximum(totals[0], totals[1])


# speedup vs baseline: 9.4328x; 1.2005x over previous
import jax
import jax.numpy as jnp
from jax.experimental import pallas as pl
from jax.experimental.pallas import tpu as pltpu

_ROWS = 32768
_COLS = 1024
_BLOCK_ROWS = 1024
_GRID = _ROWS // _BLOCK_ROWS


def _count_kernel(x_ref, y_ref, out_ref):
    out_ref[0, 0, 0] = jnp.sum((x_ref[...] > 0).astype(jnp.int32))
    out_ref[0, 0, 1] = jnp.sum((y_ref[...] > 0).astype(jnp.int32))


def kernel(x, y):
    parts = pl.pallas_call(
        _count_kernel,
        grid=(_GRID,),
        in_specs=[
            pl.BlockSpec((_BLOCK_ROWS, _COLS), lambda i: (i, 0)),
            pl.BlockSpec((_BLOCK_ROWS, _COLS), lambda i: (i, 0)),
        ],
        out_specs=pl.BlockSpec((1, 1, 2), lambda i: (i, 0, 0), memory_space=pltpu.SMEM),
        out_shape=jax.ShapeDtypeStruct((_GRID, 1, 2), jnp.int32),
        compiler_params=pltpu.CompilerParams(
            dimension_semantics=("parallel",),
        ),
    )(x, y)
    totals = parts.sum(axis=(0, 1))
    return jnp.maximum(totals[0], totals[1])
